# baseline (speedup 1.0000x reference)
import functools

import jax
import jax.numpy as jnp
from jax.experimental import pallas as pl
from jax.experimental.pallas import tpu as pltpu


_TM = 256
_TC = 256


def _mm_kernel(w_ref, x_ref, b_ref, o_ref, wb_ref, *, nc, tc, ni, half):
    i = pl.program_id(0)

    @pl.when(i < 2 * nc)
    def _():
        wb_ref[:, pl.ds(i * tc, tc)] = w_ref[...].astype(jnp.bfloat16)

    @pl.when(i >= nc)
    def _():
        xb = x_ref[...].astype(jnp.bfloat16)
        off = jnp.where(i >= nc + ni, half, 0)
        acc = jnp.dot(
            xb,
            wb_ref[:, pl.ds(off, half)],
            preferred_element_type=jnp.float32,
        )
        o_ref[...] = jnp.tanh(acc + b_ref[...])


@jax.jit
def _linear_tanh_fused(x2, w_t, b2):
    n, k = x2.shape
    m = w_t.shape[1]
    tm = min(_TM, n)
    tc = min(_TC, m)
    half = m // 2
    ni = pl.cdiv(n, tm)
    nc = pl.cdiv(half, tc)
    body = functools.partial(_mm_kernel, nc=nc, tc=tc, ni=ni, half=half)

    nw = 2 * nc

    def row(i):
        return jnp.where(
            i >= nc + ni, i - (nc + ni), jnp.maximum(i - nc, 0)
        )

    def col(i):
        return jnp.where(i >= nc + ni, 1, 0)

    return pl.pallas_call(
        body,
        out_shape=jax.ShapeDtypeStruct((n, m), jnp.float32),
        grid=(nc + 2 * ni,),
        in_specs=[
            pl.BlockSpec((k, tc), lambda i: (0, jnp.minimum(i, nw - 1))),
            pl.BlockSpec((tm, k), lambda i: (row(i), 0)),
            pl.BlockSpec((1, half), lambda i: (0, col(i))),
        ],
        out_specs=pl.BlockSpec((tm, half), lambda i: (row(i), col(i))),
        scratch_shapes=[pltpu.VMEM((k, m), jnp.bfloat16)],
        compiler_params=pltpu.CompilerParams(
            dimension_semantics=("arbitrary",),
            vmem_limit_bytes=64 * 1024 * 1024,
        ),
    )(w_t, x2, b2)


def kernel(x, w_t, b2):
    in_ch = w_t.shape[0]
    x2 = x.reshape(-1, in_ch)
    return _linear_tanh_fused(x2, w_t, b2)


# speedup vs baseline: 1.0115x; 1.0115x over previous
import functools

import jax
import jax.numpy as jnp
from jax.experimental import pallas as pl
from jax.experimental.pallas import tpu as pltpu


_TM = 256
_TC = 256


def _mm_kernel(w_ref, x_ref, b_ref, o_ref, wb_ref, *, nc, tc):
    i = pl.program_id(0)

    @pl.when(i < nc)
    def _():
        wb_ref[:, pl.ds(i * tc, tc)] = w_ref[...].astype(jnp.bfloat16)

    @pl.when(i >= nc)
    def _():
        xb = x_ref[...].astype(jnp.bfloat16)
        acc = jnp.dot(xb, wb_ref[...], preferred_element_type=jnp.float32)
        o_ref[...] = jnp.tanh(acc + b_ref[...])


@jax.jit
def _linear_tanh_fused(x2, w_t, b2):
    n, k = x2.shape
    m = w_t.shape[1]
    tm = min(_TM, n)
    tc = min(_TC, m)
    ni = pl.cdiv(n, tm)
    nc = pl.cdiv(m, tc)
    body = functools.partial(_mm_kernel, nc=nc, tc=tc)

    return pl.pallas_call(
        body,
        out_shape=jax.ShapeDtypeStruct((n, m), jnp.float32),
        grid=(nc + ni,),
        in_specs=[
            pl.BlockSpec((k, tc), lambda i: (0, jnp.minimum(i, nc - 1))),
            pl.BlockSpec((tm, k), lambda i: (jnp.maximum(i - nc, 0), 0)),
            pl.BlockSpec((1, m), lambda i: (0, 0)),
        ],
        out_specs=pl.BlockSpec(
            (tm, m), lambda i: (jnp.maximum(i - nc, 0), 0)
        ),
        scratch_shapes=[pltpu.VMEM((k, m), jnp.bfloat16)],
        compiler_params=pltpu.CompilerParams(
            dimension_semantics=("arbitrary",),
            vmem_limit_bytes=64 * 1024 * 1024,
        ),
    )(w_t, x2, b2)


def kernel(x, w_t, b2):
    in_ch = w_t.shape[0]
    x2 = x.reshape(-1, in_ch)
    return _linear_tanh_fused(x2, w_t, b2)
